# baseline (speedup 1.0000x reference)
import functools

import jax
import jax.numpy as jnp
from jax import lax
from jax.experimental import pallas as pl
from jax.experimental.pallas import tpu as pltpu, tpu_sc as plsc

N_NODES = 10000
N_PAD = 10240
E = 320000
CHUNK = 128
NW = 32
N_CHUNKS = 80
EPT = N_CHUNKS * CHUNK
E_PAD = NW * EPT
ROWS_PT = N_PAD // 16

_mesh = plsc.VectorSubcoreMesh(core_axis_name="c", subcore_axis_name="s")


def _wid(cid, sid):
    return sid * 2 + cid


def _unpack_chunk(pidx_v, i, sref, dref):
    for k in range(CHUNK // 16):
        p = pidx_v[i, pl.ds(k * 16, 16)]
        sref[pl.ds(k * 16, 16)] = lax.shift_right_logical(p, 14)
        dref[pl.ds(k * 16, 16)] = lax.bitwise_and(p, (1 << 14) - 1)


@functools.partial(
    pl.kernel,
    out_type=(
        jax.ShapeDtypeStruct((2, N_PAD), jnp.float32),
        jax.ShapeDtypeStruct((2, N_PAD), jnp.float32),
    ),
    mesh=_mesh,
    scratch_types=(
        pltpu.VMEM((N_CHUNKS, CHUNK), jnp.int32),
        pltpu.VMEM((CHUNK,), jnp.int32),
        pltpu.VMEM((CHUNK,), jnp.int32),
        pltpu.VMEM((CHUNK,), jnp.int32),
        pltpu.VMEM((CHUNK,), jnp.int32),
        pltpu.VMEM((CHUNK,), jnp.float32),
        pltpu.VMEM_SHARED((N_PAD,), jnp.float32),
        pltpu.VMEM_SHARED((N_PAD,), jnp.float32),
        pltpu.SemaphoreType.DMA,
        pltpu.SemaphoreType.DMA,
        pltpu.SemaphoreType.DMA,
        pltpu.SemaphoreType.DMA,
    ),
)
def _deg_kernel(pidx_hbm, zeros_hbm, dego_hbm, degi_hbm,
                pidx_v, sidx0, didx0, sidx1, didx1, ones_v, dego_s, degi_s,
                semo0, semi0, semo1, semi1):
    cid = lax.axis_index("c")
    sid = lax.axis_index("s")
    wid = _wid(cid, sid)

    for j in range(CHUNK // 16):
        ones_v[pl.ds(j * 16, 16)] = jnp.full((16,), 1.0, jnp.float32)

    pltpu.sync_copy(pidx_hbm.at[wid], pidx_v)

    r0 = sid * ROWS_PT
    pltpu.sync_copy(zeros_hbm.at[pl.ds(r0, ROWS_PT)], dego_s.at[pl.ds(r0, ROWS_PT)])
    pltpu.sync_copy(zeros_hbm.at[pl.ds(r0, ROWS_PT)], degi_s.at[pl.ds(r0, ROWS_PT)])
    plsc.subcore_barrier()

    @pl.loop(0, N_CHUNKS, step=2)
    def _chunk(i):
        _unpack_chunk(pidx_v, i, sidx0, didx0)
        d0 = pltpu.async_copy(ones_v, dego_s.at[sidx0], semo0, add=True)
        d1 = pltpu.async_copy(ones_v, degi_s.at[didx0], semi0, add=True)
        _unpack_chunk(pidx_v, i + 1, sidx1, didx1)
        d2 = pltpu.async_copy(ones_v, dego_s.at[sidx1], semo1, add=True)
        d3 = pltpu.async_copy(ones_v, degi_s.at[didx1], semi1, add=True)
        d0.wait()
        d1.wait()
        d2.wait()
        d3.wait()

    plsc.subcore_barrier()
    pltpu.sync_copy(dego_s.at[pl.ds(r0, ROWS_PT)], dego_hbm.at[cid, pl.ds(r0, ROWS_PT)])
    pltpu.sync_copy(degi_s.at[pl.ds(r0, ROWS_PT)], degi_hbm.at[cid, pl.ds(r0, ROWS_PT)])


@functools.partial(
    pl.kernel,
    out_type=jax.ShapeDtypeStruct((2, N_PAD, 128), jnp.float32),
    mesh=_mesh,
    scratch_types=(
        pltpu.VMEM((N_CHUNKS, CHUNK), jnp.int32),
        pltpu.VMEM((CHUNK,), jnp.int32),
        pltpu.VMEM((CHUNK,), jnp.int32),
        pltpu.VMEM((CHUNK,), jnp.int32),
        pltpu.VMEM((CHUNK,), jnp.int32),
        pltpu.VMEM((CHUNK, 128), jnp.float32),
        pltpu.VMEM((CHUNK, 128), jnp.float32),
        pltpu.VMEM_SHARED((N_PAD, 128), jnp.float32),
        pltpu.SemaphoreType.DMA,
        pltpu.SemaphoreType.DMA,
        pltpu.SemaphoreType.DMA,
    ),
)
def _prop128(h_hbm, pidx_hbm, zeros_hbm, out_hbm,
             pidx_v, sidx0, didx0, sidx1, didx1, msgs0, msgs1, acc_s,
             sem0, sem1, sems0):
    cid = lax.axis_index("c")
    sid = lax.axis_index("s")
    wid = _wid(cid, sid)
    r0 = sid * ROWS_PT

    pltpu.sync_copy(pidx_hbm.at[wid], pidx_v)
    pltpu.sync_copy(zeros_hbm.at[pl.ds(r0, ROWS_PT)], acc_s.at[pl.ds(r0, ROWS_PT)])
    plsc.subcore_barrier()

    _unpack_chunk(pidx_v, 0, sidx0, didx0)
    pltpu.async_copy(h_hbm.at[sidx0], msgs0, sem0)

    @pl.loop(0, N_CHUNKS, step=2)
    def _chunk(i):
        _unpack_chunk(pidx_v, i + 1, sidx1, didx1)
        pltpu.async_copy(h_hbm.at[sidx1], msgs1, sem1)
        pltpu.make_async_copy(h_hbm.at[sidx0], msgs0, sem0).wait()
        s0 = pltpu.async_copy(msgs0, acc_s.at[didx0], sems0, add=True)

        pltpu.make_async_copy(h_hbm.at[sidx1], msgs1, sem1).wait()
        s0.wait()

        @pl.when(i + 2 < N_CHUNKS)
        def _():
            _unpack_chunk(pidx_v, i + 2, sidx0, didx0)
            pltpu.async_copy(h_hbm.at[sidx0], msgs0, sem0)

        pltpu.sync_copy(msgs1, acc_s.at[didx1], add=True)

    plsc.subcore_barrier()
    pltpu.sync_copy(acc_s.at[pl.ds(r0, ROWS_PT)],
                    out_hbm.at[cid, pl.ds(r0, ROWS_PT)])


def _norms(dego_ref, degi_ref):
    do = dego_ref[0, :] + dego_ref[1, :]
    di = degi_ref[0, :] + degi_ref[1, :]
    ns = jnp.where(do > 0, lax.rsqrt(jnp.maximum(do, 1.0)), 0.0)
    nd = jnp.where(di > 0, lax.rsqrt(jnp.maximum(di, 1.0)), 0.0)
    return ns, nd


def _k1a_body(x_ref, w1_ref, y_ref):
    y_ref[...] = jnp.dot(x_ref[...], w1_ref[...],
                         preferred_element_type=jnp.float32)


def _k1b_body(y_ref, dego_ref, degi_ref, h1_ref):
    ns, _ = _norms(dego_ref, degi_ref)
    h1_ref[...] = y_ref[...] * ns[:, None]


def _k3_body(p_ref, dego_ref, degi_ref, b1_ref, t_ref):
    ns, nd = _norms(dego_ref, degi_ref)
    t = (p_ref[0] + p_ref[1]) * nd[:, None] + b1_ref[...]
    t_ref[...] = jnp.maximum(t, 0.0) * ns[:, None]


def _k5_body(p_ref, dego_ref, degi_ref, b2_ref, w2_ref, out_ref):
    _, nd = _norms(dego_ref, degi_ref)
    agg = p_ref[0] + p_ref[1]
    out_ref[...] = (jnp.dot(agg, w2_ref[...], preferred_element_type=jnp.float32)
                    * nd[:, None] + b2_ref[...])


_BLK = 1024
_GRID = N_PAD // _BLK


def _deg_spec():
    return pl.BlockSpec((2, _BLK), lambda i: (0, i))


def kernel(x, W1, b1, W2, b2, edge_index):
    src = edge_index[0].astype(jnp.int32)
    dst = edge_index[1].astype(jnp.int32)
    pad = N_NODES + jnp.arange(E_PAD - E, dtype=jnp.int32) % (N_PAD - N_NODES)
    src = jnp.concatenate([src, pad])
    dst = jnp.concatenate([dst, pad])
    pidx = ((src << 14) | dst).reshape(NW, N_CHUNKS, CHUNK)
    xp = jnp.zeros((N_PAD, 128), jnp.float32).at[:N_NODES].set(x)
    z1 = jnp.zeros((N_PAD,), jnp.float32)
    z128 = jnp.zeros((N_PAD, 128), jnp.float32)

    dego, degi = _deg_kernel(pidx, z1)

    y = pl.pallas_call(
        _k1a_body,
        grid=(_GRID,),
        in_specs=[
            pl.BlockSpec((_BLK, 128), lambda i: (i, 0)),
            pl.BlockSpec((128, 128), lambda i: (0, 0)),
        ],
        out_specs=pl.BlockSpec((_BLK, 128), lambda i: (i, 0)),
        out_shape=jax.ShapeDtypeStruct((N_PAD, 128), jnp.float32),
    )(xp, W1)

    h1 = pl.pallas_call(
        _k1b_body,
        grid=(_GRID,),
        in_specs=[
            pl.BlockSpec((_BLK, 128), lambda i: (i, 0)),
            _deg_spec(), _deg_spec(),
        ],
        out_specs=pl.BlockSpec((_BLK, 128), lambda i: (i, 0)),
        out_shape=jax.ShapeDtypeStruct((N_PAD, 128), jnp.float32),
    )(y, dego, degi)

    parts1 = _prop128(h1, pidx, z128)

    t = pl.pallas_call(
        _k3_body,
        grid=(_GRID,),
        in_specs=[
            pl.BlockSpec((2, _BLK, 128), lambda i: (0, i, 0)),
            _deg_spec(), _deg_spec(),
            pl.BlockSpec((1, 128), lambda i: (0, 0)),
        ],
        out_specs=pl.BlockSpec((_BLK, 128), lambda i: (i, 0)),
        out_shape=jax.ShapeDtypeStruct((N_PAD, 128), jnp.float32),
    )(parts1, dego, degi, b1.reshape(1, 128))

    parts2 = _prop128(t, pidx, z128)

    out = pl.pallas_call(
        _k5_body,
        grid=(_GRID,),
        in_specs=[
            pl.BlockSpec((2, _BLK, 128), lambda i: (0, i, 0)),
            _deg_spec(), _deg_spec(),
            pl.BlockSpec((1, 40), lambda i: (0, 0)),
            pl.BlockSpec((128, 40), lambda i: (0, 0)),
        ],
        out_specs=pl.BlockSpec((_BLK, 40), lambda i: (i, 0)),
        out_shape=jax.ShapeDtypeStruct((N_PAD, 40), jnp.float32),
    )(parts2, dego, degi, b2.reshape(1, 40), W2)

    return out[:N_NODES]


# speedup vs baseline: 1.0278x; 1.0278x over previous
import functools

import jax
import jax.numpy as jnp
from jax import lax
from jax.experimental import pallas as pl
from jax.experimental.pallas import tpu as pltpu, tpu_sc as plsc

N_NODES = 10000
N_PAD = 10240
E = 320000
CHUNK = 128
NW = 32
N_CHUNKS = 80
EPT = N_CHUNKS * CHUNK
E_PAD = NW * EPT
ROWS_PT = N_PAD // 16

_mesh = plsc.VectorSubcoreMesh(core_axis_name="c", subcore_axis_name="s")


def _wid(cid, sid):
    return sid * 2 + cid


def _unpack_chunk(pidx_v, i, sref, dref):
    for k in range(CHUNK // 16):
        p = pidx_v[i, pl.ds(k * 16, 16)]
        sref[pl.ds(k * 16, 16)] = lax.shift_right_logical(p, 14)
        dref[pl.ds(k * 16, 16)] = lax.bitwise_and(p, (1 << 14) - 1)


@functools.partial(
    pl.kernel,
    out_type=(
        jax.ShapeDtypeStruct((2, N_PAD), jnp.float32),
        jax.ShapeDtypeStruct((2, N_PAD), jnp.float32),
    ),
    mesh=_mesh,
    scratch_types=(
        pltpu.VMEM((N_CHUNKS, CHUNK), jnp.int32),
        pltpu.VMEM((CHUNK,), jnp.int32),
        pltpu.VMEM((CHUNK,), jnp.int32),
        pltpu.VMEM((CHUNK,), jnp.int32),
        pltpu.VMEM((CHUNK,), jnp.int32),
        pltpu.VMEM((CHUNK,), jnp.float32),
        pltpu.VMEM_SHARED((N_PAD,), jnp.float32),
        pltpu.VMEM_SHARED((N_PAD,), jnp.float32),
        pltpu.SemaphoreType.DMA,
        pltpu.SemaphoreType.DMA,
        pltpu.SemaphoreType.DMA,
        pltpu.SemaphoreType.DMA,
    ),
)
def _deg_kernel(pidx_hbm, zeros_hbm, dego_hbm, degi_hbm,
                pidx_v, sidx0, didx0, sidx1, didx1, ones_v, dego_s, degi_s,
                semo0, semi0, semo1, semi1):
    cid = lax.axis_index("c")
    sid = lax.axis_index("s")
    wid = _wid(cid, sid)

    for j in range(CHUNK // 16):
        ones_v[pl.ds(j * 16, 16)] = jnp.full((16,), 1.0, jnp.float32)

    pltpu.sync_copy(pidx_hbm.at[wid], pidx_v)

    r0 = sid * ROWS_PT
    pltpu.sync_copy(zeros_hbm.at[pl.ds(r0, ROWS_PT)], dego_s.at[pl.ds(r0, ROWS_PT)])
    pltpu.sync_copy(zeros_hbm.at[pl.ds(r0, ROWS_PT)], degi_s.at[pl.ds(r0, ROWS_PT)])
    plsc.subcore_barrier()

    @pl.loop(0, N_CHUNKS, step=2)
    def _chunk(i):
        _unpack_chunk(pidx_v, i, sidx0, didx0)
        d0 = pltpu.async_copy(ones_v, dego_s.at[sidx0], semo0, add=True)
        d1 = pltpu.async_copy(ones_v, degi_s.at[didx0], semi0, add=True)
        _unpack_chunk(pidx_v, i + 1, sidx1, didx1)
        d2 = pltpu.async_copy(ones_v, dego_s.at[sidx1], semo1, add=True)
        d3 = pltpu.async_copy(ones_v, degi_s.at[didx1], semi1, add=True)
        d0.wait()
        d1.wait()
        d2.wait()
        d3.wait()

    plsc.subcore_barrier()
    pltpu.sync_copy(dego_s.at[pl.ds(r0, ROWS_PT)], dego_hbm.at[cid, pl.ds(r0, ROWS_PT)])
    pltpu.sync_copy(degi_s.at[pl.ds(r0, ROWS_PT)], degi_hbm.at[cid, pl.ds(r0, ROWS_PT)])


@functools.partial(
    pl.kernel,
    out_type=jax.ShapeDtypeStruct((2, N_PAD, 128), jnp.float32),
    mesh=_mesh,
    scratch_types=(
        pltpu.VMEM((N_CHUNKS, CHUNK), jnp.int32),
        pltpu.VMEM((CHUNK,), jnp.int32),
        pltpu.VMEM((CHUNK,), jnp.int32),
        pltpu.VMEM((CHUNK,), jnp.int32),
        pltpu.VMEM((CHUNK,), jnp.int32),
        pltpu.VMEM((CHUNK, 128), jnp.float32),
        pltpu.VMEM((CHUNK, 128), jnp.float32),
        pltpu.VMEM_SHARED((N_PAD, 128), jnp.float32),
        pltpu.SemaphoreType.DMA,
        pltpu.SemaphoreType.DMA,
    ),
)
def _prop128(h_hbm, pidx_hbm, zeros_hbm, out_hbm,
             pidx_v, sidx0, didx0, sidx1, didx1, msgs0, msgs1, acc_s,
             sem0, sem1):
    cid = lax.axis_index("c")
    sid = lax.axis_index("s")
    wid = _wid(cid, sid)
    r0 = sid * ROWS_PT

    pltpu.sync_copy(pidx_hbm.at[wid], pidx_v)
    pltpu.sync_copy(zeros_hbm.at[pl.ds(r0, ROWS_PT)], acc_s.at[pl.ds(r0, ROWS_PT)])
    plsc.subcore_barrier()

    _unpack_chunk(pidx_v, 0, sidx0, didx0)
    pltpu.async_copy(h_hbm.at[sidx0], msgs0, sem0)

    @pl.loop(0, N_CHUNKS, step=2)
    def _chunk(i):
        _unpack_chunk(pidx_v, i + 1, sidx1, didx1)
        pltpu.async_copy(h_hbm.at[sidx1], msgs1, sem1)
        pltpu.make_async_copy(h_hbm.at[sidx0], msgs0, sem0).wait()
        pltpu.sync_copy(msgs0, acc_s.at[didx0], add=True)

        @pl.when(i + 2 < N_CHUNKS)
        def _():
            _unpack_chunk(pidx_v, i + 2, sidx0, didx0)
            pltpu.async_copy(h_hbm.at[sidx0], msgs0, sem0)

        pltpu.make_async_copy(h_hbm.at[sidx1], msgs1, sem1).wait()
        pltpu.sync_copy(msgs1, acc_s.at[didx1], add=True)

    plsc.subcore_barrier()
    pltpu.sync_copy(acc_s.at[pl.ds(r0, ROWS_PT)],
                    out_hbm.at[cid, pl.ds(r0, ROWS_PT)])


def _norms(dego_ref, degi_ref):
    do = dego_ref[0, :] + dego_ref[1, :]
    di = degi_ref[0, :] + degi_ref[1, :]
    ns = jnp.where(do > 0, lax.rsqrt(jnp.maximum(do, 1.0)), 0.0)
    nd = jnp.where(di > 0, lax.rsqrt(jnp.maximum(di, 1.0)), 0.0)
    return ns, nd


def _k1a_body(x_ref, w1_ref, y_ref):
    y_ref[...] = jnp.dot(x_ref[...], w1_ref[...],
                         preferred_element_type=jnp.float32)


def _k1b_body(y_ref, dego_ref, degi_ref, h1_ref):
    ns, _ = _norms(dego_ref, degi_ref)
    h1_ref[...] = y_ref[...] * ns[:, None]


def _k3_body(p_ref, dego_ref, degi_ref, b1_ref, t_ref):
    ns, nd = _norms(dego_ref, degi_ref)
    t = (p_ref[0] + p_ref[1]) * nd[:, None] + b1_ref[...]
    t_ref[...] = jnp.maximum(t, 0.0) * ns[:, None]


def _k5_body(p_ref, dego_ref, degi_ref, b2_ref, w2_ref, out_ref):
    _, nd = _norms(dego_ref, degi_ref)
    agg = p_ref[0] + p_ref[1]
    out_ref[...] = (jnp.dot(agg, w2_ref[...], preferred_element_type=jnp.float32)
                    * nd[:, None] + b2_ref[...])


_BLK = 1024
_GRID = N_PAD // _BLK


def _deg_spec():
    return pl.BlockSpec((2, _BLK), lambda i: (0, i))


def kernel(x, W1, b1, W2, b2, edge_index):
    src = edge_index[0].astype(jnp.int32)
    dst = edge_index[1].astype(jnp.int32)
    pad = N_NODES + jnp.arange(E_PAD - E, dtype=jnp.int32) % (N_PAD - N_NODES)
    src = jnp.concatenate([src, pad])
    dst = jnp.concatenate([dst, pad])
    pidx = ((src << 14) | dst).reshape(NW, N_CHUNKS, CHUNK)
    xp = jnp.zeros((N_PAD, 128), jnp.float32).at[:N_NODES].set(x)
    z1 = jnp.zeros((N_PAD,), jnp.float32)
    z128 = jnp.zeros((N_PAD, 128), jnp.float32)

    dego, degi = _deg_kernel(pidx, z1)

    y = pl.pallas_call(
        _k1a_body,
        grid=(_GRID,),
        in_specs=[
            pl.BlockSpec((_BLK, 128), lambda i: (i, 0)),
            pl.BlockSpec((128, 128), lambda i: (0, 0)),
        ],
        out_specs=pl.BlockSpec((_BLK, 128), lambda i: (i, 0)),
        out_shape=jax.ShapeDtypeStruct((N_PAD, 128), jnp.float32),
    )(xp, W1)

    h1 = pl.pallas_call(
        _k1b_body,
        grid=(_GRID,),
        in_specs=[
            pl.BlockSpec((_BLK, 128), lambda i: (i, 0)),
            _deg_spec(), _deg_spec(),
        ],
        out_specs=pl.BlockSpec((_BLK, 128), lambda i: (i, 0)),
        out_shape=jax.ShapeDtypeStruct((N_PAD, 128), jnp.float32),
    )(y, dego, degi)

    parts1 = _prop128(h1, pidx, z128)

    t = pl.pallas_call(
        _k3_body,
        grid=(_GRID,),
        in_specs=[
            pl.BlockSpec((2, _BLK, 128), lambda i: (0, i, 0)),
            _deg_spec(), _deg_spec(),
            pl.BlockSpec((1, 128), lambda i: (0, 0)),
        ],
        out_specs=pl.BlockSpec((_BLK, 128), lambda i: (i, 0)),
        out_shape=jax.ShapeDtypeStruct((N_PAD, 128), jnp.float32),
    )(parts1, dego, degi, b1.reshape(1, 128))

    parts2 = _prop128(t, pidx, z128)

    out = pl.pallas_call(
        _k5_body,
        grid=(_GRID,),
        in_specs=[
            pl.BlockSpec((2, _BLK, 128), lambda i: (0, i, 0)),
            _deg_spec(), _deg_spec(),
            pl.BlockSpec((1, 40), lambda i: (0, 0)),
            pl.BlockSpec((128, 40), lambda i: (0, 0)),
        ],
        out_specs=pl.BlockSpec((_BLK, 40), lambda i: (i, 0)),
        out_shape=jax.ShapeDtypeStruct((N_PAD, 40), jnp.float32),
    )(parts2, dego, degi, b2.reshape(1, 40), W2)

    return out[:N_NODES]
